# baseline (speedup 1.0000x reference)
import functools

import jax
import jax.numpy as jnp
from jax import lax
from jax.experimental import pallas as pl
from jax.experimental.pallas import tpu as pltpu

_B, _T, _D = 4, 2048, 768
_E = 8
_EP = 128
_BM = 512
_NEG = -1e30


def _moe_block(x_ref, gw_ref, gb_ref, w_ref, b_ref, o_ref):
    x = x_ref[...]
    gw = gw_ref[...]
    gb = gb_ref[...]
    logits = lax.dot_general(x, gw, (((1,), (1,)), ((), ())),
                             preferred_element_type=jnp.float32) + gb
    eidx = lax.broadcasted_iota(jnp.int32, logits.shape, 1)
    m1 = jnp.max(logits, axis=1, keepdims=True)
    a1 = jnp.min(jnp.where(logits >= m1, eidx, _EP), axis=1, keepdims=True)
    mask1 = eidx == a1
    l2 = jnp.where(mask1, _NEG, logits)
    m2 = jnp.max(l2, axis=1, keepdims=True)
    a2 = jnp.min(jnp.where(l2 >= m2, eidx, _EP), axis=1, keepdims=True)
    mask2 = eidx == a2
    combine = (jnp.where(mask1, jax.nn.sigmoid(m1), 0.0)
               + jnp.where(mask2, jax.nn.sigmoid(m2), 0.0))
    acc = lax.dot_general(combine, b_ref[...], (((1,), (0,)), ((), ())),
                          preferred_element_type=jnp.float32)
    xb = x.astype(jnp.bfloat16)
    for e in range(_E):
        ye = lax.dot_general(xb, w_ref[e], (((1,), (1,)), ((), ())),
                             preferred_element_type=jnp.float32)
        acc = acc + combine[:, e:e + 1] * ye
    o_ref[...] = acc


def kernel(inputs, gate_W, gate_b, expert_W, expert_b):
    n = _B * _T
    x = inputs.reshape(n, _D)
    w_bf16 = expert_W.astype(jnp.bfloat16)
    gw_p = jnp.zeros((_EP, _D), jnp.float32).at[:_E].set(gate_W)
    gb_p = jnp.full((1, _EP), _NEG, jnp.float32).at[0, :_E].set(gate_b)
    b_p = jnp.zeros((_EP, _D), jnp.float32).at[:_E].set(expert_b)
    out = pl.pallas_call(
        _moe_block,
        grid=(n // _BM,),
        in_specs=[
            pl.BlockSpec((_BM, _D), lambda i: (i, 0)),
            pl.BlockSpec((_EP, _D), lambda i: (0, 0)),
            pl.BlockSpec((1, _EP), lambda i: (0, 0)),
            pl.BlockSpec((_E, _D, _D), lambda i: (0, 0, 0)),
            pl.BlockSpec((_EP, _D), lambda i: (0, 0)),
        ],
        out_specs=pl.BlockSpec((_BM, _D), lambda i: (i, 0)),
        out_shape=jax.ShapeDtypeStruct((n, _D), jnp.float32),
        compiler_params=pltpu.CompilerParams(
            dimension_semantics=("arbitrary",),
        ),
    )(x, gw_p, gb_p, w_bf16, b_p)
    return out.reshape(_B, _T, _D)


# speedup vs baseline: 1.0490x; 1.0490x over previous
import functools

import jax
import jax.numpy as jnp
from jax import lax
from jax.experimental import pallas as pl
from jax.experimental.pallas import tpu as pltpu

_B, _T, _D = 4, 2048, 768
_E = 8
_EP = 128
_BM = 1024
_NEG = -1e30


def _moe_block(x_ref, gw_ref, gb_ref, w_ref, b_ref, o_ref):
    x = x_ref[...]
    gw = gw_ref[...]
    gb = gb_ref[...]
    logits = lax.dot_general(x, gw, (((1,), (1,)), ((), ())),
                             preferred_element_type=jnp.float32) + gb
    eidx = lax.broadcasted_iota(jnp.int32, logits.shape, 1)
    m1 = jnp.max(logits, axis=1, keepdims=True)
    a1 = jnp.min(jnp.where(logits >= m1, eidx, _EP), axis=1, keepdims=True)
    mask1 = eidx == a1
    l2 = jnp.where(mask1, _NEG, logits)
    m2 = jnp.max(l2, axis=1, keepdims=True)
    a2 = jnp.min(jnp.where(l2 >= m2, eidx, _EP), axis=1, keepdims=True)
    mask2 = eidx == a2
    combine = (jnp.where(mask1, jax.nn.sigmoid(m1), 0.0)
               + jnp.where(mask2, jax.nn.sigmoid(m2), 0.0))
    acc = lax.dot_general(combine, b_ref[...], (((1,), (0,)), ((), ())),
                          preferred_element_type=jnp.float32)
    xb = x.astype(jnp.bfloat16)
    for e in range(_E):
        ye = lax.dot_general(xb, w_ref[e], (((1,), (1,)), ((), ())),
                             preferred_element_type=jnp.float32)
        acc = acc + combine[:, e:e + 1] * ye
    o_ref[...] = acc


def kernel(inputs, gate_W, gate_b, expert_W, expert_b):
    n = _B * _T
    x = inputs.reshape(n, _D)
    w_bf16 = expert_W.astype(jnp.bfloat16)
    gw_p = jnp.zeros((_EP, _D), jnp.float32).at[:_E].set(gate_W)
    gb_p = jnp.full((1, _EP), _NEG, jnp.float32).at[0, :_E].set(gate_b)
    b_p = jnp.zeros((_EP, _D), jnp.float32).at[:_E].set(expert_b)
    out = pl.pallas_call(
        _moe_block,
        grid=(n // _BM,),
        in_specs=[
            pl.BlockSpec((_BM, _D), lambda i: (i, 0)),
            pl.BlockSpec((_EP, _D), lambda i: (0, 0)),
            pl.BlockSpec((1, _EP), lambda i: (0, 0)),
            pl.BlockSpec((_E, _D, _D), lambda i: (0, 0, 0)),
            pl.BlockSpec((_EP, _D), lambda i: (0, 0)),
        ],
        out_specs=pl.BlockSpec((_BM, _D), lambda i: (i, 0)),
        out_shape=jax.ShapeDtypeStruct((n, _D), jnp.float32),
        compiler_params=pltpu.CompilerParams(
            dimension_semantics=("arbitrary",),
        ),
    )(x, gw_p, gb_p, w_bf16, b_p)
    return out.reshape(_B, _T, _D)
